# baseline (speedup 1.0000x reference)
import jax
import jax.numpy as jnp
from jax.experimental import pallas as pl

_ROWS = 100000
_COLS = 512
_BLOCK = 2000


def _body(x_ref, o_ref):
    o_ref[...] = x_ref[...]

    @pl.when(pl.program_id(0) == 0)
    def _overwrite():
        two = jnp.full((1, _COLS), 2.0, jnp.float32)
        o_ref[pl.ds(1, 1), :] = two
        o_ref[pl.ds(3, 1), :] = two


def kernel(x):
    return pl.pallas_call(
        _body,
        grid=(_ROWS // _BLOCK,),
        in_specs=[pl.BlockSpec((_BLOCK, _COLS), lambda i: (i, 0))],
        out_specs=pl.BlockSpec((_BLOCK, _COLS), lambda i: (i, 0)),
        out_shape=jax.ShapeDtypeStruct((_ROWS, _COLS), jnp.float32),
    )(x)
